# Initial kernel scaffold; baseline (speedup 1.0000x reference)
import functools

import jax
import jax.numpy as jnp
from jax import lax
from jax.experimental import pallas as pl
from jax.experimental.pallas import tpu as pltpu
from jax.experimental.pallas import tpu_sc as plsc

DIM = 64
SZ = 512
CH = 16
P = 128


def _tables_body(xy_ref, xz_ref, yz_ref, w_ref, out_ref):
    for p, ref in enumerate((xy_ref, xz_ref, yz_ref)):
        a = ref[...]
        wp = w_ref[:, p * DIM:(p + 1) * DIM]
        r = lax.dot_general(a, wp, (((0,), (1,)), ((), ())),
                            preferred_element_type=jnp.float32)
        out_ref[p] = r


def _build_tables(xy, xz, yz, lin_w):
    grid = SZ // CH
    bs_plane = pl.BlockSpec((DIM, CH, SZ), lambda j: (0, j, 0))
    bs_w = pl.BlockSpec((DIM, 3 * DIM), lambda j: (0, 0))
    bs_out = pl.BlockSpec((3, CH, SZ, DIM), lambda j: (0, j, 0, 0))
    return pl.pallas_call(
        _tables_body,
        grid=(grid,),
        in_specs=[bs_plane, bs_plane, bs_plane, bs_w],
        out_specs=bs_out,
        out_shape=jax.ShapeDtypeStruct((3, SZ, SZ, DIM), jnp.float32),
    )(xy, xz, yz, lin_w)


def _axis_setup(g):
    ixf = ((g + 1.0) * float(SZ) - 1.0) * 0.5
    ixf = jnp.clip(ixf, -64.0, float(SZ) + 64.0)
    t = ixf.astype(jnp.int32)
    tf = t.astype(jnp.float32)
    c0 = jnp.where(tf > ixf, t - 1, t)
    c1 = c0 + 1
    w1 = ixf - c0.astype(jnp.float32)
    w0 = 1.0 - w1
    w0m = jnp.where((c0 >= 0) & (c0 <= SZ - 1), w0, 0.0)
    w1m = jnp.where((c1 >= 0) & (c1 <= SZ - 1), w1, 0.0)
    return jnp.clip(c0, 0, SZ - 1), jnp.clip(c1, 0, SZ - 1), w0m, w1m


def _sc_sample(gx, gy, gz, table, bias):
    n = gx.shape[0]
    info = plsc.get_sparse_core_info()
    nw = info.num_cores * info.num_subcores
    npw = n // nw
    nblk = npw // P
    mesh = plsc.VectorSubcoreMesh(core_axis_name="c", subcore_axis_name="s")

    @functools.partial(
        pl.kernel, mesh=mesh,
        out_type=jax.ShapeDtypeStruct((n, DIM), jnp.float32),
        scratch_types=[
            pltpu.VMEM((P,), jnp.float32),
            pltpu.VMEM((P,), jnp.float32),
            pltpu.VMEM((P,), jnp.float32),
            pltpu.VMEM((12, P), jnp.int32),
            pltpu.VMEM((12, P), jnp.float32),
            pltpu.VMEM((12, P, DIM), jnp.float32),
            pltpu.VMEM((P, DIM), jnp.float32),
            pltpu.VMEM((DIM,), jnp.float32),
            pltpu.SemaphoreType.DMA,
        ],
    )
    def body(gx_h, gy_h, gz_h, tab_h, b_h, out_h,
             gx_v, gy_v, gz_v, idx_v, w_v, rows_v, acc_v, b_v, sem):
        wid = lax.axis_index("s") * info.num_cores + lax.axis_index("c")
        base0 = wid * npw
        pltpu.sync_copy(b_h, b_v)
        bias_regs = [b_v[pl.ds(k * 16, 16)] for k in range(4)]

        def blk_body(blk, carry):
            base = base0 + blk * P
            pltpu.sync_copy(gx_h.at[pl.ds(base, P)], gx_v)
            pltpu.sync_copy(gy_h.at[pl.ds(base, P)], gy_v)
            pltpu.sync_copy(gz_h.at[pl.ds(base, P)], gz_v)

            def grp_body(i, c2):
                s = pl.ds(i * 16, 16)
                ax_ = _axis_setup(gx_v[s])
                ay_ = _axis_setup(gy_v[s])
                az_ = _axis_setup(gz_v[s])
                c = 0
                for p, (axA, axB) in enumerate(((ax_, ay_), (ax_, az_), (ay_, az_))):
                    a0, a1, aw0, aw1 = axA
                    b0, b1, bw0, bw1 = axB
                    pb = p * SZ * SZ
                    for (acoord, aw) in ((a0, aw0), (a1, aw1)):
                        for (bcoord, bw) in ((b0, bw0), (b1, bw1)):
                            idx_v[c, s] = pb + bcoord * SZ + acoord
                            w_v[c, s] = aw * bw
                            c += 1
                return c2
            lax.fori_loop(0, P // 16, grp_body, 0)

            handles = [pltpu.async_copy(tab_h.at[idx_v.at[c]], rows_v.at[c], sem)
                       for c in range(12)]
            for h in handles:
                h.wait()

            def pt_body(m, c2):
                accs = list(bias_regs)
                for c in range(12):
                    wv = jnp.full((16,), w_v[c, m], jnp.float32)
                    for k in range(4):
                        accs[k] = accs[k] + wv * rows_v[c, m, pl.ds(k * 16, 16)]
                for k in range(4):
                    acc_v[m, pl.ds(k * 16, 16)] = accs[k]
                return c2
            lax.fori_loop(0, P, pt_body, 0)

            pltpu.sync_copy(acc_v, out_h.at[pl.ds(base, P)])
            return carry
        lax.fori_loop(0, nblk, blk_body, 0)

    return body(gx, gy, gz, table, bias)


def kernel(x, xy, xz, yz, lin_w, lin_b):
    table = _build_tables(xy, xz, yz, lin_w).reshape(3 * SZ * SZ, DIM)
    gx, gy, gz = x[:, 0], x[:, 1], x[:, 2]
    return _sc_sample(gx, gy, gz, table, lin_b)


# speedup vs baseline: 4.8420x; 4.8420x over previous
import functools

import jax
import jax.numpy as jnp
from jax import lax
from jax.experimental import pallas as pl
from jax.experimental.pallas import tpu as pltpu
from jax.experimental.pallas import tpu_sc as plsc

DIM = 64
SZ = 512
CH = 16
P = 128


def _tables_body(xy_ref, xz_ref, yz_ref, w_ref, out_ref):
    for p, ref in enumerate((xy_ref, xz_ref, yz_ref)):
        a = ref[...]
        wp = w_ref[:, p * DIM:(p + 1) * DIM]
        r = lax.dot_general(a, wp, (((0,), (1,)), ((), ())),
                            preferred_element_type=jnp.float32)
        out_ref[p] = r


def _build_tables(xy, xz, yz, lin_w):
    grid = SZ // CH
    bs_plane = pl.BlockSpec((DIM, CH, SZ), lambda j: (0, j, 0))
    bs_w = pl.BlockSpec((DIM, 3 * DIM), lambda j: (0, 0))
    bs_out = pl.BlockSpec((3, CH, SZ, DIM), lambda j: (0, j, 0, 0))
    return pl.pallas_call(
        _tables_body,
        grid=(grid,),
        in_specs=[bs_plane, bs_plane, bs_plane, bs_w],
        out_specs=bs_out,
        out_shape=jax.ShapeDtypeStruct((3, SZ, SZ, DIM), jnp.float32),
    )(xy, xz, yz, lin_w)


def _axis_setup(g):
    ixf = ((g + 1.0) * float(SZ) - 1.0) * 0.5
    ixf = jnp.clip(ixf, -64.0, float(SZ) + 64.0)
    t = ixf.astype(jnp.int32)
    tf = t.astype(jnp.float32)
    c0 = jnp.where(tf > ixf, t - 1, t)
    c1 = c0 + 1
    w1 = ixf - c0.astype(jnp.float32)
    w0 = 1.0 - w1
    w0m = jnp.where((c0 >= 0) & (c0 <= SZ - 1), w0, 0.0)
    w1m = jnp.where((c1 >= 0) & (c1 <= SZ - 1), w1, 0.0)
    return jnp.clip(c0, 0, SZ - 1), jnp.clip(c1, 0, SZ - 1), w0m, w1m


def _sc_sample(gx, gy, gz, table, bias):
    n = gx.shape[0]
    info = plsc.get_sparse_core_info()
    nw = info.num_cores * info.num_subcores
    npw = n // nw
    nblk = npw // P
    mesh = plsc.VectorSubcoreMesh(core_axis_name="c", subcore_axis_name="s")

    @functools.partial(
        pl.kernel, mesh=mesh,
        out_type=jax.ShapeDtypeStruct((n, DIM), jnp.float32),
        compiler_params=pltpu.CompilerParams(use_tc_tiling_on_sc=False),
        scratch_types=[
            pltpu.VMEM((P,), jnp.float32),
            pltpu.VMEM((P,), jnp.float32),
            pltpu.VMEM((P,), jnp.float32),
            pltpu.VMEM((12, P), jnp.int32),
            pltpu.VMEM((12, P), jnp.float32),
            pltpu.VMEM((12, P, DIM), jnp.float32),
            pltpu.VMEM((P, DIM), jnp.float32),
            pltpu.VMEM((DIM,), jnp.float32),
            pltpu.SemaphoreType.DMA,
        ],
    )
    def body(gx_h, gy_h, gz_h, tab_h, b_h, out_h,
             gx_v, gy_v, gz_v, idx_v, w_v, rows_v, acc_v, b_v, sem):
        wid = lax.axis_index("s") * info.num_cores + lax.axis_index("c")
        base0 = wid * npw
        pltpu.sync_copy(b_h, b_v)
        bias_regs = [b_v[pl.ds(k * 16, 16)] for k in range(4)]

        def blk_body(blk, carry):
            base = base0 + blk * P
            pltpu.sync_copy(gx_h.at[pl.ds(base, P)], gx_v)
            pltpu.sync_copy(gy_h.at[pl.ds(base, P)], gy_v)
            pltpu.sync_copy(gz_h.at[pl.ds(base, P)], gz_v)

            def grp_body(i, c2):
                s = pl.ds(i * 16, 16)
                ax_ = _axis_setup(gx_v[s])
                ay_ = _axis_setup(gy_v[s])
                az_ = _axis_setup(gz_v[s])
                c = 0
                for p, (axA, axB) in enumerate(((ax_, ay_), (ax_, az_), (ay_, az_))):
                    a0, a1, aw0, aw1 = axA
                    b0, b1, bw0, bw1 = axB
                    pb = p * SZ * SZ
                    for (acoord, aw) in ((a0, aw0), (a1, aw1)):
                        for (bcoord, bw) in ((b0, bw0), (b1, bw1)):
                            idx_v[c, s] = pb + bcoord * SZ + acoord
                            w_v[c, s] = aw * bw
                            c += 1
                return c2
            lax.fori_loop(0, P // 16, grp_body, 0)

            handles = [pltpu.async_copy(tab_h.at[idx_v.at[c]], rows_v.at[c], sem)
                       for c in range(12)]
            for h in handles:
                h.wait()

            def ptg_body(g, c2):
                sg = pl.ds(g * 16, 16)
                wvecs = [w_v[c, sg] for c in range(12)]
                rbase = g * 16
                for j in range(16):
                    m = rbase + j
                    accs = list(bias_regs)
                    for c in range(12):
                        wv = jnp.full((16,), wvecs[c][j], jnp.float32)
                        for k in range(4):
                            accs[k] = accs[k] + wv * rows_v[c, m, pl.ds(k * 16, 16)]
                    for k in range(4):
                        acc_v[m, pl.ds(k * 16, 16)] = accs[k]
                return c2
            lax.fori_loop(0, P // 16, ptg_body, 0)

            pltpu.sync_copy(acc_v, out_h.at[pl.ds(base, P)])
            return carry
        lax.fori_loop(0, nblk, blk_body, 0)

    return body(gx, gy, gz, table, bias)


def kernel(x, xy, xz, yz, lin_w, lin_b):
    table = _build_tables(xy, xz, yz, lin_w).reshape(3 * SZ * SZ, DIM)
    gx, gy, gz = x[:, 0], x[:, 1], x[:, 2]
    return _sc_sample(gx, gy, gz, table, lin_b)
